# baseline (speedup 1.0000x reference)
import functools

import jax
import jax.numpy as jnp
from jax import lax
from jax.experimental import pallas as pl
from jax.experimental.pallas import tpu as pltpu
from jax.experimental.pallas import tpu_sc as plsc

_N = 10000
_E_DIM = 128
_N_EDGES = 320000
_MAX_EDGES = 200
_BOOST = 0.0

_PROJ_BLK = 1000
_DOT_BLK = 2000
_GATHER_CHUNK = 400


def _proj_body(x_ref, w_ref, o_ref):
    o_ref[...] = lax.dot_general(
        x_ref[...], w_ref[...], (((1,), (1,)), ((), ())),
        preferred_element_type=jnp.float32)


def _project(table, w):
    n = table.shape[0]
    return pl.pallas_call(
        _proj_body,
        grid=(n // _PROJ_BLK,),
        in_specs=[
            pl.BlockSpec((_PROJ_BLK, _E_DIM), lambda g: (g, 0)),
            pl.BlockSpec((_E_DIM, _E_DIM), lambda g: (0, 0)),
        ],
        out_specs=pl.BlockSpec((_PROJ_BLK, _E_DIM), lambda g: (g, 0)),
        out_shape=jax.ShapeDtypeStruct((n, _E_DIM), jnp.float32),
    )(table, w)


def _make_gather_kernel():
    info = plsc.get_sparse_core_info()
    nw = info.num_cores * info.num_subcores
    b_per_w = _N_EDGES // nw
    n_chunks = b_per_w // _GATHER_CHUNK
    mesh = plsc.VectorSubcoreMesh(core_axis_name="c", subcore_axis_name="s")

    out_t = jax.ShapeDtypeStruct((_N_EDGES, _E_DIM), jnp.float32)

    @functools.partial(
        pl.kernel, mesh=mesh,
        out_type=(out_t, out_t, out_t),
        scratch_types=[
            pltpu.VMEM((_GATHER_CHUNK,), jnp.int32),
            pltpu.VMEM((_GATHER_CHUNK, _E_DIM), jnp.float32),
            pltpu.SemaphoreType.DMA,
        ],
    )
    def gather_kernel(ktab, ptab, qtab, si, pi, oi, ko, po, qo,
                      idx_v, rows_v, sem):
        wid = lax.axis_index("s") * info.num_cores + lax.axis_index("c")
        base = wid * b_per_w
        for tab, idx, out in ((ktab, si, ko), (ptab, pi, po), (qtab, oi, qo)):
            for ci in range(n_chunks):
                off = base + ci * _GATHER_CHUNK
                pltpu.sync_copy(idx.at[pl.ds(off, _GATHER_CHUNK)], idx_v)
                pltpu.async_copy(tab.at[idx_v], rows_v, sem).wait()
                pltpu.sync_copy(rows_v, out.at[pl.ds(off, _GATHER_CHUNK)])

    return gather_kernel


def _dots_body(k_ref, p_ref, q_ref, o_ref):
    x = (k_ref[...] * p_ref[...]) * q_ref[...]
    acc = x[:, 0:8]
    for a in range(1, 16):
        acc = acc + x[:, 8 * a:8 * a + 8]
    h1 = acc[:, 0:4] + acc[:, 4:8]
    h2 = h1[:, 0:2] + h1[:, 2:4]
    s = h2[:, 0:1] + h2[:, 1:2]
    o_ref[...] = s * jnp.float32(0.0078125)


def _dots(kg, pg, qg):
    return pl.pallas_call(
        _dots_body,
        grid=(_N_EDGES // _DOT_BLK,),
        in_specs=[
            pl.BlockSpec((_DOT_BLK, _E_DIM), lambda g: (g, 0)),
            pl.BlockSpec((_DOT_BLK, _E_DIM), lambda g: (g, 0)),
            pl.BlockSpec((_DOT_BLK, _E_DIM), lambda g: (g, 0)),
        ],
        out_specs=pl.BlockSpec((_DOT_BLK, 1), lambda g: (g, 0)),
        out_shape=jax.ShapeDtypeStruct((_N_EDGES, 1), jnp.float32),
    )(kg, pg, qg)


def kernel(embeddings, relations, tokeys, toqueries, si, pi, oi):
    kt = _project(embeddings, tokeys)
    qt = _project(embeddings, toqueries)
    kg, pg, qg = _make_gather_kernel()(kt, relations, qt, si, pi, oi)
    dots = _dots(kg, pg, qg).reshape(_N_EDGES)

    order = jnp.argsort(-dots)
    sorted_dots = jnp.take(dots, order, axis=0)
    probs = jax.nn.sigmoid(sorted_dots + _BOOST)
    bern = jax.random.bernoulli(jax.random.key(1), probs)
    cs = jnp.cumsum(bern.astype(jnp.int32)) <= _MAX_EDGES
    mask_sorted = jnp.logical_and(bern, cs)
    mask = jnp.zeros((_N_EDGES,), jnp.bool_).at[order].set(mask_sorted)
    return dots, mask


# speedup vs baseline: 1.6651x; 1.6651x over previous
import functools

import jax
import jax.numpy as jnp
from jax import lax
from jax.experimental import pallas as pl
from jax.experimental.pallas import tpu as pltpu
from jax.experimental.pallas import tpu_sc as plsc

_N = 10000
_E_DIM = 128
_N_EDGES = 320000
_MAX_EDGES = 200
_BOOST = 0.0

_PROJ_BLK = 1000
_DOT_BLK = 2000
_GATHER_CHUNK = 400


def _proj_body(x_ref, w_ref, o_ref):
    o_ref[...] = lax.dot_general(
        x_ref[...], w_ref[...], (((1,), (1,)), ((), ())),
        preferred_element_type=jnp.float32)


def _project(table, w):
    n = table.shape[0]
    return pl.pallas_call(
        _proj_body,
        grid=(n // _PROJ_BLK,),
        in_specs=[
            pl.BlockSpec((_PROJ_BLK, _E_DIM), lambda g: (g, 0)),
            pl.BlockSpec((_E_DIM, _E_DIM), lambda g: (0, 0)),
        ],
        out_specs=pl.BlockSpec((_PROJ_BLK, _E_DIM), lambda g: (g, 0)),
        out_shape=jax.ShapeDtypeStruct((n, _E_DIM), jnp.float32),
    )(table, w)


def _make_gather_kernel():
    info = plsc.get_sparse_core_info()
    nw = info.num_cores * info.num_subcores
    b_per_w = _N_EDGES // nw
    n_chunks = b_per_w // _GATHER_CHUNK
    mesh = plsc.VectorSubcoreMesh(core_axis_name="c", subcore_axis_name="s")

    out_t = jax.ShapeDtypeStruct((_N_EDGES, _E_DIM), jnp.float32)

    @functools.partial(
        pl.kernel, mesh=mesh,
        out_type=(out_t, out_t, out_t),
        scratch_types=[
            pltpu.VMEM((_GATHER_CHUNK,), jnp.int32),
            pltpu.VMEM((_GATHER_CHUNK,), jnp.int32),
            pltpu.VMEM((_GATHER_CHUNK, _E_DIM), jnp.float32),
            pltpu.VMEM((_GATHER_CHUNK, _E_DIM), jnp.float32),
            pltpu.SemaphoreType.DMA,
            pltpu.SemaphoreType.DMA,
        ],
    )
    def gather_kernel(ktab, ptab, qtab, si, pi, oi, ko, po, qo,
                      idx_v0, idx_v1, rows_v0, rows_v1, sem0, sem1):
        wid = lax.axis_index("s") * info.num_cores + lax.axis_index("c")
        base = wid * b_per_w
        idx_bufs = (idx_v0, idx_v1)
        row_bufs = (rows_v0, rows_v1)
        sems = (sem0, sem1)
        chunks = [
            (tab, idx, out, ci * _GATHER_CHUNK)
            for tab, idx, out in ((ktab, si, ko), (ptab, pi, po), (qtab, oi, qo))
            for ci in range(n_chunks)
        ]
        handles = [None, None]
        for j, (tab, idx, out, off) in enumerate(chunks):
            p = j % 2
            pltpu.sync_copy(idx.at[pl.ds(base + off, _GATHER_CHUNK)],
                            idx_bufs[p])
            handles[p] = pltpu.async_copy(tab.at[idx_bufs[p]], row_bufs[p],
                                          sems[p])
            if j > 0:
                ptab_, pidx_, pout_, poff_ = chunks[j - 1]
                handles[1 - p].wait()
                pltpu.sync_copy(row_bufs[1 - p],
                                pout_.at[pl.ds(base + poff_, _GATHER_CHUNK)])
        ltab_, lidx_, lout_, loff_ = chunks[-1]
        lastp = (len(chunks) - 1) % 2
        handles[lastp].wait()
        pltpu.sync_copy(row_bufs[lastp],
                        lout_.at[pl.ds(base + loff_, _GATHER_CHUNK)])

    return gather_kernel


def _dots_body(k_ref, p_ref, q_ref, o_ref):
    x = (k_ref[...] * p_ref[...]) * q_ref[...]
    acc = x[:, 0:8]
    for a in range(1, 16):
        acc = acc + x[:, 8 * a:8 * a + 8]
    h1 = acc[:, 0:4] + acc[:, 4:8]
    h2 = h1[:, 0:2] + h1[:, 2:4]
    s = h2[:, 0:1] + h2[:, 1:2]
    o_ref[...] = s * jnp.float32(0.0078125)


def _dots(kg, pg, qg):
    return pl.pallas_call(
        _dots_body,
        grid=(_N_EDGES // _DOT_BLK,),
        in_specs=[
            pl.BlockSpec((_DOT_BLK, _E_DIM), lambda g: (g, 0)),
            pl.BlockSpec((_DOT_BLK, _E_DIM), lambda g: (g, 0)),
            pl.BlockSpec((_DOT_BLK, _E_DIM), lambda g: (g, 0)),
        ],
        out_specs=pl.BlockSpec((_DOT_BLK, 1), lambda g: (g, 0)),
        out_shape=jax.ShapeDtypeStruct((_N_EDGES, 1), jnp.float32),
    )(kg, pg, qg)


def kernel(embeddings, relations, tokeys, toqueries, si, pi, oi):
    kt = _project(embeddings, tokeys)
    qt = _project(embeddings, toqueries)
    kg, pg, qg = _make_gather_kernel()(kt, relations, qt, si, pi, oi)
    dots = _dots(kg, pg, qg).reshape(_N_EDGES)

    order = jnp.argsort(-dots)
    sorted_dots = jnp.take(dots, order, axis=0)
    probs = jax.nn.sigmoid(sorted_dots + _BOOST)
    bern = jax.random.bernoulli(jax.random.key(1), probs)
    cs = jnp.cumsum(bern.astype(jnp.int32)) <= _MAX_EDGES
    mask_sorted = jnp.logical_and(bern, cs)
    inv = jnp.argsort(order)
    mask = jnp.take(mask_sorted, inv, axis=0)
    return dots, mask


# speedup vs baseline: 1.8279x; 1.0978x over previous
import functools

import jax
import jax.numpy as jnp
from jax import lax
from jax.experimental import pallas as pl
from jax.experimental.pallas import tpu as pltpu
from jax.experimental.pallas import tpu_sc as plsc

_N = 10000
_E_DIM = 128
_N_EDGES = 320000
_MAX_EDGES = 200
_BOOST = 0.0

_PROJ_BLK = 1000
_DOT_BLK = 2000
_GATHER_CHUNK = 400


def _proj_body(x_ref, w_ref, o_ref):
    o_ref[...] = lax.dot_general(
        x_ref[...], w_ref[...], (((1,), (1,)), ((), ())),
        preferred_element_type=jnp.float32)


def _project(table, w):
    n = table.shape[0]
    return pl.pallas_call(
        _proj_body,
        grid=(n // _PROJ_BLK,),
        in_specs=[
            pl.BlockSpec((_PROJ_BLK, _E_DIM), lambda g: (g, 0)),
            pl.BlockSpec((_E_DIM, _E_DIM), lambda g: (0, 0)),
        ],
        out_specs=pl.BlockSpec((_PROJ_BLK, _E_DIM), lambda g: (g, 0)),
        out_shape=jax.ShapeDtypeStruct((n, _E_DIM), jnp.float32),
    )(table, w)


def _make_gather_kernel(n_edges, chunk):
    info = plsc.get_sparse_core_info()
    nw = info.num_cores * info.num_subcores
    b_per_w = n_edges // nw
    n_chunks = b_per_w // chunk
    mesh = plsc.VectorSubcoreMesh(core_axis_name="c", subcore_axis_name="s")

    out_t = jax.ShapeDtypeStruct((n_edges, _E_DIM), jnp.float32)

    @functools.partial(
        pl.kernel, mesh=mesh,
        out_type=(out_t, out_t, out_t),
        scratch_types=[
            pltpu.VMEM((chunk,), jnp.int32),
            pltpu.VMEM((chunk,), jnp.int32),
            pltpu.VMEM((chunk, _E_DIM), jnp.float32),
            pltpu.VMEM((chunk, _E_DIM), jnp.float32),
            pltpu.SemaphoreType.DMA,
            pltpu.SemaphoreType.DMA,
        ],
    )
    def gather_kernel(ktab, ptab, qtab, si, pi, oi, ko, po, qo,
                      idx_v0, idx_v1, rows_v0, rows_v1, sem0, sem1):
        wid = lax.axis_index("s") * info.num_cores + lax.axis_index("c")
        base = wid * b_per_w
        idx_bufs = (idx_v0, idx_v1)
        row_bufs = (rows_v0, rows_v1)
        sems = (sem0, sem1)
        chunks = [
            (tab, idx, out, ci * chunk)
            for tab, idx, out in ((ktab, si, ko), (ptab, pi, po), (qtab, oi, qo))
            for ci in range(n_chunks)
        ]
        handles = [None, None]
        for j, (tab, idx, out, off) in enumerate(chunks):
            p = j % 2
            pltpu.sync_copy(idx.at[pl.ds(base + off, chunk)], idx_bufs[p])
            handles[p] = pltpu.async_copy(tab.at[idx_bufs[p]], row_bufs[p],
                                          sems[p])
            if j > 0:
                ptab_, pidx_, pout_, poff_ = chunks[j - 1]
                handles[1 - p].wait()
                pltpu.sync_copy(row_bufs[1 - p],
                                pout_.at[pl.ds(base + poff_, chunk)])
        ltab_, lidx_, lout_, loff_ = chunks[-1]
        lastp = (len(chunks) - 1) % 2
        handles[lastp].wait()
        pltpu.sync_copy(row_bufs[lastp],
                        lout_.at[pl.ds(base + loff_, chunk)])

    return gather_kernel


def _dots_body(k_ref, p_ref, q_ref, o_ref):
    x = (k_ref[...] * p_ref[...]) * q_ref[...]
    acc = x[:, 0:8]
    for a in range(1, 16):
        acc = acc + x[:, 8 * a:8 * a + 8]
    h1 = acc[:, 0:4] + acc[:, 4:8]
    h2 = h1[:, 0:2] + h1[:, 2:4]
    s = h2[:, 0:1] + h2[:, 1:2]
    o_ref[...] = s * jnp.float32(0.0078125)


def _dots(kg, pg, qg):
    n = kg.shape[0]
    return pl.pallas_call(
        _dots_body,
        grid=(n // _DOT_BLK,),
        in_specs=[
            pl.BlockSpec((_DOT_BLK, _E_DIM), lambda g: (g, 0)),
            pl.BlockSpec((_DOT_BLK, _E_DIM), lambda g: (g, 0)),
            pl.BlockSpec((_DOT_BLK, _E_DIM), lambda g: (g, 0)),
        ],
        out_specs=pl.BlockSpec((_DOT_BLK, 1), lambda g: (g, 0)),
        out_shape=jax.ShapeDtypeStruct((n, 1), jnp.float32),
    )(kg, pg, qg)


def kernel(embeddings, relations, tokeys, toqueries, si, pi, oi):
    kt = _project(embeddings, tokeys)
    qt = _project(embeddings, toqueries)
    half = _N_EDGES // 2
    gather = _make_gather_kernel(half, 200)
    parts = []
    for lo in (0, half):
        kg, pg, qg = gather(kt, relations, qt,
                            si[lo:lo + half], pi[lo:lo + half],
                            oi[lo:lo + half])
        parts.append(_dots(kg, pg, qg).reshape(half))
    dots = jnp.concatenate(parts)

    order = jnp.argsort(-dots)
    sorted_dots = jnp.take(dots, order, axis=0)
    probs = jax.nn.sigmoid(sorted_dots + _BOOST)
    bern = jax.random.bernoulli(jax.random.key(1), probs)
    cs = jnp.cumsum(bern.astype(jnp.int32)) <= _MAX_EDGES
    mask_sorted = jnp.logical_and(bern, cs)
    inv = jnp.argsort(order)
    mask = jnp.take(mask_sorted, inv, axis=0)
    return dots, mask
